# baseline (speedup 1.0000x reference)
import functools

import jax
import jax.numpy as jnp
from jax.experimental import pallas as pl
from jax.experimental.pallas import tpu as pltpu

_K = 100
_BM = 256


def _body(x_ref, we_ref, be_ref, wd_ref, bd_ref, o_ref):
    x = x_ref[...] - bd_ref[...]
    pre = jnp.dot(x, we_ref[...], preferred_element_type=jnp.float32)
    y = jnp.maximum(pre + be_ref[...], 0.0)
    bits = jax.lax.bitcast_convert_type(y, jnp.int32)
    bm = x.shape[0]
    d_sae = y.shape[1]

    ones = jnp.ones((d_sae, 1), jnp.bfloat16)
    one_b = jnp.bfloat16(1)
    zero_b = jnp.bfloat16(0)

    def count(m):
        sel = jnp.where(m, one_b, zero_b)
        return jnp.dot(sel, ones, preferred_element_type=jnp.float32)

    y16 = jax.lax.bitcast_convert_type(
        jnp.bitwise_and(bits, jnp.int32(-65536)), jnp.float32
    ).astype(jnp.bfloat16)
    lo = (jnp.bitwise_xor(bits, 0x8000) & 0xFFFF).astype(jnp.int16)

    kf = jnp.float32(_K)

    def step1(i, t):
        cand = jnp.bitwise_or(t, jax.lax.shift_left(1, 14 - i))
        cand_b = jax.lax.bitcast_convert_type(cand.astype(jnp.int16), jnp.bfloat16)
        cnt = count(y16 >= cand_b)
        return jnp.where(cnt >= kf, cand, t)

    t1 = jax.lax.fori_loop(0, 15, step1, jnp.zeros((bm, 1), jnp.int32))
    t1_b = jax.lax.bitcast_convert_type(t1.astype(jnp.int16), jnp.bfloat16)
    n_gt = count(y16 > t1_b)
    meq = y16 == t1_b

    def step2(i, t):
        cand = jnp.bitwise_or(t, jax.lax.shift_left(1, 15 - i))
        cand16 = jnp.bitwise_xor(cand, 0x8000).astype(jnp.int16)
        cnt = n_gt + count((lo >= cand16) & meq)
        return jnp.where(cnt >= kf, cand, t)

    u = jax.lax.fori_loop(0, 16, step2, jnp.zeros((bm, 1), jnp.int32))
    thr = jnp.bitwise_or(jax.lax.shift_left(t1, 16), u)

    enc = jnp.where(bits >= thr, y, 0.0).astype(jnp.bfloat16)
    o_ref[...] = (
        jnp.dot(enc, wd_ref[...], preferred_element_type=jnp.float32) + bd_ref[...]
    )


@jax.jit
def kernel(x, W_enc, b_enc, W_dec, b_dec):
    B, d_in = x.shape
    d_sae = W_enc.shape[1]
    be = b_enc.reshape(1, d_sae)
    bd = b_dec.reshape(1, d_in)
    grid = (B // _BM,)
    return pl.pallas_call(
        _body,
        grid=grid,
        in_specs=[
            pl.BlockSpec((_BM, d_in), lambda i: (i, 0)),
            pl.BlockSpec((d_in, d_sae), lambda i: (0, 0)),
            pl.BlockSpec((1, d_sae), lambda i: (0, 0)),
            pl.BlockSpec((d_sae, d_in), lambda i: (0, 0)),
            pl.BlockSpec((1, d_in), lambda i: (0, 0)),
        ],
        out_specs=pl.BlockSpec((_BM, d_in), lambda i: (i, 0)),
        out_shape=jax.ShapeDtypeStruct((B, d_in), jnp.float32),
    )(x, W_enc, be, W_dec.astype(jnp.bfloat16), bd)


# speedup vs baseline: 1.8179x; 1.8179x over previous
import functools

import jax
import jax.numpy as jnp
from jax.experimental import pallas as pl
from jax.experimental.pallas import tpu as pltpu

_K = 100
_BM = 256


def _body(x_ref, we_ref, be_ref, wd_ref, bd_ref, o_ref):
    xm = (x_ref[...] - bd_ref[...]).astype(jnp.bfloat16)
    pre = jnp.dot(xm, we_ref[...], preferred_element_type=jnp.float32)
    y = jnp.maximum(pre + be_ref[...], 0.0)
    bits = jax.lax.bitcast_convert_type(y, jnp.int32)

    def step(i, t):
        cand = jnp.bitwise_or(t, jax.lax.shift_left(1, 30 - i))
        cnt = jnp.sum((bits >= cand).astype(jnp.float32), axis=1, keepdims=True)
        return jnp.where(cnt >= float(_K), cand, t)

    t = jax.lax.fori_loop(0, 31, step, jnp.zeros((x_ref.shape[0], 1), jnp.int32))
    enc = jnp.where(bits >= t, y, 0.0).astype(jnp.bfloat16)
    o_ref[...] = (
        jnp.dot(enc, wd_ref[...], preferred_element_type=jnp.float32) + bd_ref[...]
    )


@jax.jit
def kernel(x, W_enc, b_enc, W_dec, b_dec):
    B, d_in = x.shape
    d_sae = W_enc.shape[1]
    be = b_enc.reshape(1, d_sae)
    bd = b_dec.reshape(1, d_in)
    grid = (B // _BM,)
    return pl.pallas_call(
        _body,
        grid=grid,
        in_specs=[
            pl.BlockSpec((_BM, d_in), lambda i: (i, 0)),
            pl.BlockSpec((d_in, d_sae), lambda i: (0, 0)),
            pl.BlockSpec((1, d_sae), lambda i: (0, 0)),
            pl.BlockSpec((d_sae, d_in), lambda i: (0, 0)),
            pl.BlockSpec((1, d_in), lambda i: (0, 0)),
        ],
        out_specs=pl.BlockSpec((_BM, d_in), lambda i: (i, 0)),
        out_shape=jax.ShapeDtypeStruct((B, d_in), jnp.float32),
    )(x, W_enc.astype(jnp.bfloat16), be, W_dec.astype(jnp.bfloat16), bd)


# speedup vs baseline: 1.9631x; 1.0799x over previous
import functools

import jax
import jax.numpy as jnp
from jax.experimental import pallas as pl
from jax.experimental.pallas import tpu as pltpu

_K = 100
_BM = 256


def _tree_count(m_bool):
    s = jnp.where(m_bool, jnp.bfloat16(1), jnp.bfloat16(0))
    while s.shape[1] > 128:
        h = s.shape[1] // 2
        s = s[:, :h] + s[:, h:]
    return jnp.sum(s.astype(jnp.float32), axis=1, keepdims=True)


def _body(x_ref, we_ref, be_ref, wd_ref, bd_ref, o_ref):
    xm = (x_ref[...] - bd_ref[...]).astype(jnp.bfloat16)
    pre = jnp.dot(xm, we_ref[...], preferred_element_type=jnp.float32)
    y = jnp.maximum(pre + be_ref[...], 0.0)
    bits = jax.lax.bitcast_convert_type(y, jnp.int32)
    bm = y.shape[0]
    kf = jnp.float32(_K)

    y16 = jax.lax.bitcast_convert_type(
        jnp.bitwise_and(bits, jnp.int32(-65536)), jnp.float32
    ).astype(jnp.bfloat16)
    lo = (jnp.bitwise_xor(bits, 0x8000) & 0xFFFF).astype(jnp.int16)

    def step1(i, t):
        cand = jnp.bitwise_or(t, jax.lax.shift_left(1, 14 - i))
        cand_b = jax.lax.bitcast_convert_type(cand.astype(jnp.int16), jnp.bfloat16)
        cnt = _tree_count(y16 >= cand_b)
        return jnp.where(cnt >= kf, cand, t)

    t1 = jax.lax.fori_loop(0, 15, step1, jnp.zeros((bm, 1), jnp.int32))
    t1_b = jax.lax.bitcast_convert_type(t1.astype(jnp.int16), jnp.bfloat16)
    n_gt = _tree_count(y16 > t1_b)
    meq = y16 == t1_b

    def step2(i, t):
        cand = jnp.bitwise_or(t, jax.lax.shift_left(1, 15 - i))
        cand16 = jnp.bitwise_xor(cand, 0x8000).astype(jnp.int16)
        cnt = n_gt + _tree_count((lo >= cand16) & meq)
        return jnp.where(cnt >= kf, cand, t)

    u = jax.lax.fori_loop(0, 16, step2, jnp.zeros((bm, 1), jnp.int32))
    thr = jnp.bitwise_or(jax.lax.shift_left(t1, 16), u)

    enc = jnp.where(bits >= thr, y, 0.0).astype(jnp.bfloat16)
    o_ref[...] = (
        jnp.dot(enc, wd_ref[...], preferred_element_type=jnp.float32) + bd_ref[...]
    )


@jax.jit
def kernel(x, W_enc, b_enc, W_dec, b_dec):
    B, d_in = x.shape
    d_sae = W_enc.shape[1]
    be = b_enc.reshape(1, d_sae)
    bd = b_dec.reshape(1, d_in)
    grid = (B // _BM,)
    return pl.pallas_call(
        _body,
        grid=grid,
        in_specs=[
            pl.BlockSpec((_BM, d_in), lambda i: (i, 0)),
            pl.BlockSpec((d_in, d_sae), lambda i: (0, 0)),
            pl.BlockSpec((1, d_sae), lambda i: (0, 0)),
            pl.BlockSpec((d_sae, d_in), lambda i: (0, 0)),
            pl.BlockSpec((1, d_in), lambda i: (0, 0)),
        ],
        out_specs=pl.BlockSpec((_BM, d_in), lambda i: (i, 0)),
        out_shape=jax.ShapeDtypeStruct((B, d_in), jnp.float32),
    )(x, W_enc.astype(jnp.bfloat16), be, W_dec.astype(jnp.bfloat16), bd)
